# Initial kernel scaffold; baseline (speedup 1.0000x reference)
import functools

import jax
import jax.numpy as jnp
from jax import lax
from jax.experimental import pallas as pl
from jax.experimental.pallas import tpu as pltpu
from jax.experimental.pallas import tpu_sc as plsc

_N = 10000
_E = 320000
_D = 128
_H = 128
_Z = 64
_MAXN = 28
_G = 64
_L = 3

_R = 2000
_NC = 2
_NS = 16
_NW = _NC * _NS
_EPW = _E // _NW
_CH = 80
_NCH = _EPW // _CH
_RPT = _N // _NS
_RC = 125
_NRC = _RPT // _RC

_f32 = jnp.float32


def _enc_body(x_ref, w_ref, b_ref, mw_ref, mb_ref, h_ref, hw_ref):
    h = jnp.maximum(
        jnp.dot(x_ref[...], w_ref[...], preferred_element_type=_f32) + b_ref[...],
        0.0)
    h_ref[...] = h
    hw_ref[...] = jnp.dot(h, mw_ref[...], preferred_element_type=_f32) + mb_ref[...]


def _enc(x, w, b, mw, mb):
    return pl.pallas_call(
        _enc_body,
        grid=(_N // _R,),
        in_specs=[
            pl.BlockSpec((_R, _D), lambda i: (i, 0)),
            pl.BlockSpec((_D, _H), lambda i: (0, 0)),
            pl.BlockSpec((1, _H), lambda i: (0, 0)),
            pl.BlockSpec((_H, _H), lambda i: (0, 0)),
            pl.BlockSpec((1, _H), lambda i: (0, 0)),
        ],
        out_specs=[
            pl.BlockSpec((_R, _H), lambda i: (i, 0)),
            pl.BlockSpec((_R, _H), lambda i: (i, 0)),
        ],
        out_shape=[jax.ShapeDtypeStruct((_N, _H), _f32)] * 2,
    )(x, w, b, mw, mb)


def _sc_scatter(hw, src, dst, zrows):
    mesh = plsc.VectorSubcoreMesh(core_axis_name="c", subcore_axis_name="s")

    @functools.partial(
        pl.kernel,
        mesh=mesh,
        out_type=[jax.ShapeDtypeStruct((_N, _H), _f32)] * 2,
        scratch_types=[
            pltpu.VMEM((_CH,), jnp.int32),
            pltpu.VMEM((_CH,), jnp.int32),
            pltpu.VMEM((_CH, _H), _f32),
            pltpu.VMEM((_RC, _H), _f32),
            pltpu.VMEM_SHARED((_N, _H), _f32),
            pltpu.SemaphoreType.DMA,
        ],
    )
    def k(hw_hbm, src_hbm, dst_hbm, z_hbm, out0, out1,
          src_v, dst_v, rows_v, stage_v, acc, sem):
        c = lax.axis_index("c")
        s = lax.axis_index("s")
        row0 = s * _RPT
        pltpu.sync_copy(z_hbm, stage_v)
        for i in range(_NRC):
            pltpu.sync_copy(stage_v, acc.at[pl.ds(row0 + i * _RC, _RC)])
        plsc.subcore_barrier()

        base = (s * _NC + c) * _EPW

        def body(j, carry):
            off = pl.multiple_of(base + j * _CH, 8)
            pltpu.sync_copy(src_hbm.at[pl.ds(off, _CH)], src_v)
            pltpu.sync_copy(dst_hbm.at[pl.ds(off, _CH)], dst_v)
            pltpu.async_copy(hw_hbm.at[src_v], rows_v, sem).wait()
            pltpu.sync_copy(rows_v, acc.at[dst_v], add=True)
            return carry

        lax.fori_loop(0, _NCH, body, 0)
        plsc.subcore_barrier()

        @pl.when(c == 0)
        def _():
            for i in range(_NRC):
                r = row0 + i * _RC
                pltpu.sync_copy(acc.at[pl.ds(r, _RC)], stage_v)
                pltpu.sync_copy(stage_v, out0.at[pl.ds(r, _RC)])

        @pl.when(c == 1)
        def _():
            for i in range(_NRC):
                r = row0 + i * _RC
                pltpu.sync_copy(acc.at[pl.ds(r, _RC)], stage_v)
                pltpu.sync_copy(stage_v, out1.at[pl.ds(r, _RC)])

    return k(hw, src, dst, zrows)


def _upd_mid_body(h_ref, a0_ref, a1_ref, wt_ref, wb_ref, b_ref, mw_ref, mb_ref,
                  hn_ref, hw_ref):
    a = a0_ref[...] + a1_ref[...]
    hn = jnp.maximum(
        jnp.dot(h_ref[...], wt_ref[...], preferred_element_type=_f32)
        + jnp.dot(a, wb_ref[...], preferred_element_type=_f32)
        + b_ref[...], 0.0)
    hn_ref[...] = hn
    hw_ref[...] = jnp.dot(hn, mw_ref[...], preferred_element_type=_f32) + mb_ref[...]


def _upd_mid(h, a0, a1, wt, wb, b, mw, mb):
    return pl.pallas_call(
        _upd_mid_body,
        grid=(_N // _R,),
        in_specs=[
            pl.BlockSpec((_R, _H), lambda i: (i, 0)),
            pl.BlockSpec((_R, _H), lambda i: (i, 0)),
            pl.BlockSpec((_R, _H), lambda i: (i, 0)),
            pl.BlockSpec((_H, _H), lambda i: (0, 0)),
            pl.BlockSpec((_H, _H), lambda i: (0, 0)),
            pl.BlockSpec((1, _H), lambda i: (0, 0)),
            pl.BlockSpec((_H, _H), lambda i: (0, 0)),
            pl.BlockSpec((1, _H), lambda i: (0, 0)),
        ],
        out_specs=[
            pl.BlockSpec((_R, _H), lambda i: (i, 0)),
            pl.BlockSpec((_R, _H), lambda i: (i, 0)),
        ],
        out_shape=[jax.ShapeDtypeStruct((_N, _H), _f32)] * 2,
    )(h, a0, a1, wt, wb, b, mw, mb)


def _upd_last_body(h_ref, a0_ref, a1_ref, bt_ref, wt_ref, wb_ref, b_ref, ge_ref):
    a = a0_ref[...] + a1_ref[...]
    hn = jnp.maximum(
        jnp.dot(h_ref[...], wt_ref[...], preferred_element_type=_f32)
        + jnp.dot(a, wb_ref[...], preferred_element_type=_f32)
        + b_ref[...], 0.0)
    gi = lax.broadcasted_iota(jnp.int32, (_R, _G), 1)
    mask = (bt_ref[...] == gi).astype(_f32)
    part = lax.dot_general(mask, hn, (((0,), (0,)), ((), ())),
                           preferred_element_type=_f32)

    @pl.when(pl.program_id(0) == 0)
    def _():
        ge_ref[...] = jnp.zeros_like(ge_ref)

    ge_ref[...] += part


def _upd_last(h, a0, a1, bt, wt, wb, b):
    return pl.pallas_call(
        _upd_last_body,
        grid=(_N // _R,),
        in_specs=[
            pl.BlockSpec((_R, _H), lambda i: (i, 0)),
            pl.BlockSpec((_R, _H), lambda i: (i, 0)),
            pl.BlockSpec((_R, _H), lambda i: (i, 0)),
            pl.BlockSpec((_R, 1), lambda i: (i, 0)),
            pl.BlockSpec((_H, _H), lambda i: (0, 0)),
            pl.BlockSpec((_H, _H), lambda i: (0, 0)),
            pl.BlockSpec((1, _H), lambda i: (0, 0)),
        ],
        out_specs=pl.BlockSpec((_G, _H), lambda i: (0, 0)),
        out_shape=jax.ShapeDtypeStruct((_G, _H), _f32),
    )(h, a0, a1, bt, wt, wb, b)


def _dec_body(ge_ref, mw_ref, mb_ref, lw_ref, lb_ref, eps_ref,
              d1w_ref, d1b_ref, d2w_ref, d2b_ref, nw_ref, nb_ref,
              zm_ref, zl_ref, emb_ref):
    ge = ge_ref[...]
    zm = jnp.dot(ge, mw_ref[...], preferred_element_type=_f32) + mb_ref[...]
    zl = jnp.dot(ge, lw_ref[...], preferred_element_type=_f32) + lb_ref[...]
    zm_ref[...] = zm
    zl_ref[...] = zl
    z = zm + eps_ref[...] * jnp.exp(0.5 * zl)
    g1 = jnp.maximum(
        jnp.dot(z, d1w_ref[...], preferred_element_type=_f32) + d1b_ref[...], 0.0)
    g2 = jnp.maximum(
        jnp.dot(g1, d2w_ref[...], preferred_element_type=_f32) + d2b_ref[...], 0.0)
    emb_ref[...] = jnp.maximum(
        jnp.dot(g2, nw_ref[...], preferred_element_type=_f32) + nb_ref[...], 0.0)


def _full_spec(shape):
    return pl.BlockSpec(shape, lambda: tuple(0 for _ in shape))


def _dec(ge, mw, mb, lw, lb, eps, d1w, d1b, d2w, d2b, nw, nb):
    return pl.pallas_call(
        _dec_body,
        in_specs=[
            _full_spec((_G, _H)), _full_spec((_H, _Z)), _full_spec((1, _Z)),
            _full_spec((_H, _Z)), _full_spec((1, _Z)), _full_spec((_G, _Z)),
            _full_spec((_Z, _H)), _full_spec((1, _H)), _full_spec((_H, _H)),
            _full_spec((1, _H)), _full_spec((_H, _H * _MAXN)),
            _full_spec((1, _H * _MAXN)),
        ],
        out_specs=[_full_spec((_G, _Z)), _full_spec((_G, _Z)),
                   _full_spec((_G, _H * _MAXN))],
        out_shape=[
            jax.ShapeDtypeStruct((_G, _Z), _f32),
            jax.ShapeDtypeStruct((_G, _Z), _f32),
            jax.ShapeDtypeStruct((_G, _H * _MAXN), _f32),
        ],
    )(ge, mw, mb, lw, lb, eps, d1w, d1b, d2w, d2b, nw, nb)


def _nmm_body(e_ref, w_ref, b_ref, o_ref):
    o_ref[...] = (jnp.dot(e_ref[...], w_ref[...], preferred_element_type=_f32)
                  + b_ref[...])


def _nmm(emb2, wcat, bcat):
    rows = _G * _MAXN
    cols = 3 * _H
    return pl.pallas_call(
        _nmm_body,
        in_specs=[_full_spec((rows, _H)), _full_spec((_H, cols)),
                  _full_spec((1, cols))],
        out_specs=_full_spec((rows, cols)),
        out_shape=jax.ShapeDtypeStruct((rows, cols), _f32),
    )(emb2, wcat, bcat)


def _pair_body(u_ref, v_ref, w2t_ref, b2_ref, o_ref):
    u = u_ref[...]
    v = v_ref[...]
    w2t = w2t_ref[...]
    b2 = b2_ref[...]
    lanes = lax.broadcasted_iota(jnp.int32, (1, _MAXN), 1)
    for a in range(_MAXN):
        h1 = jnp.maximum(u[a:a + 1, :] + v, 0.0)
        h2 = jnp.maximum(v[a:a + 1, :] + u, 0.0)
        r1 = lax.dot_general(w2t, h1, (((1,), (1,)), ((), ())),
                             preferred_element_type=_f32)
        r2 = lax.dot_general(w2t, h2, (((1,), (1,)), ((), ())),
                             preferred_element_type=_f32)
        row = (r1 + r2) * 0.5 + b2
        o_ref[a:a + 1, :] = jnp.where(lanes == a, -1e9, row)


def _pair(u, v, w2t, b2):
    return pl.pallas_call(
        _pair_body,
        grid=(_G,),
        in_specs=[
            pl.BlockSpec((None, _MAXN, _H), lambda g: (g, 0, 0)),
            pl.BlockSpec((None, _MAXN, _H), lambda g: (g, 0, 0)),
            pl.BlockSpec((1, _H), lambda g: (0, 0)),
            pl.BlockSpec((1, 1), lambda g: (0, 0)),
        ],
        out_specs=pl.BlockSpec((None, _MAXN, _MAXN), lambda g: (g, 0, 0)),
        out_shape=jax.ShapeDtypeStruct((_G, _MAXN, _MAXN), _f32),
    )(u, v, w2t, b2)


def kernel(x, edge_index, batch, enc_in_W, enc_in_b, msg_W, msg_b, upd_W, upd_b,
           mean_W, mean_b, logvar_W, logvar_b, dec1_W, dec1_b, dec2_W, dec2_b,
           nemb_W, nemb_b, nfeat_W, nfeat_b, edge1_W, edge1_b, edge2_W, edge2_b):
    row = lambda v: v.reshape(1, -1)
    src = edge_index[0]
    dst = edge_index[1]
    zrows = jnp.zeros((_RC, _H), _f32)

    h, hw = _enc(x, enc_in_W, row(enc_in_b), msg_W[0], row(msg_b[0]))

    ge = None
    for l in range(_L):
        a0, a1 = _sc_scatter(hw, src, dst, zrows)
        wt = upd_W[l][:_H]
        wb = upd_W[l][_H:]
        if l + 1 < _L:
            h, hw = _upd_mid(h, a0, a1, wt, wb, row(upd_b[l]),
                             msg_W[l + 1], row(msg_b[l + 1]))
        else:
            ge = _upd_last(h, a0, a1, batch.reshape(_N, 1), wt, wb,
                           row(upd_b[l]))

    eps = jax.random.normal(jax.random.key(42), (_G, _Z), _f32)
    z_mean, z_logvar, emb_flat = _dec(
        ge, mean_W, row(mean_b), logvar_W, row(logvar_b), eps,
        dec1_W, row(dec1_b), dec2_W, row(dec2_b), nemb_W, row(nemb_b))

    emb2 = emb_flat.reshape(_G * _MAXN, _H)
    wcat = jnp.concatenate([nfeat_W, edge1_W[:_H], edge1_W[_H:]], axis=1)
    bcat = jnp.concatenate(
        [nfeat_b, edge1_b, jnp.zeros((_H,), _f32)]).reshape(1, 3 * _H)
    out = _nmm(emb2, wcat, bcat)

    node_features = out[:, :_H].reshape(_G, _MAXN, _D)
    u = out[:, _H:2 * _H].reshape(_G, _MAXN, _H)
    v = out[:, 2 * _H:].reshape(_G, _MAXN, _H)

    adj_logits = _pair(u, v, edge2_W.reshape(1, _H), edge2_b.reshape(1, 1))
    return (node_features, adj_logits, z_mean, z_logvar)


# speedup vs baseline: 4.6552x; 4.6552x over previous
import functools

import jax
import jax.numpy as jnp
from jax import lax
from jax.experimental import pallas as pl
from jax.experimental.pallas import tpu as pltpu
from jax.experimental.pallas import tpu_sc as plsc

_N = 10000
_E = 320000
_D = 128
_H = 128
_Z = 64
_MAXN = 28
_G = 64
_L = 3

_R = 2000
_NC = 2
_NS = 16
_NW = _NC * _NS
_EPW = _E // _NW
_CH = 80
_NCH = _EPW // _CH
_NP = 10240
_RPT = _NP // _NS
_RC = 128
_NRC = _RPT // _RC

_f32 = jnp.float32


def _enc_body(x_ref, w_ref, b_ref, mw_ref, mb_ref, h_ref, hw_ref):
    h = jnp.maximum(
        jnp.dot(x_ref[...], w_ref[...], preferred_element_type=_f32) + b_ref[...],
        0.0)
    h_ref[...] = h
    hw_ref[...] = jnp.dot(h, mw_ref[...], preferred_element_type=_f32) + mb_ref[...]


def _enc(x, w, b, mw, mb):
    return pl.pallas_call(
        _enc_body,
        grid=(_N // _R,),
        in_specs=[
            pl.BlockSpec((_R, _D), lambda i: (i, 0)),
            pl.BlockSpec((_D, _H), lambda i: (0, 0)),
            pl.BlockSpec((1, _H), lambda i: (0, 0)),
            pl.BlockSpec((_H, _H), lambda i: (0, 0)),
            pl.BlockSpec((1, _H), lambda i: (0, 0)),
        ],
        out_specs=[
            pl.BlockSpec((_R, _H), lambda i: (i, 0)),
            pl.BlockSpec((_R, _H), lambda i: (i, 0)),
        ],
        out_shape=[jax.ShapeDtypeStruct((_N, _H), _f32)] * 2,
    )(x, w, b, mw, mb)


def _sc_scatter(hw, src, dst, zrows):
    mesh = plsc.VectorSubcoreMesh(core_axis_name="c", subcore_axis_name="s")

    @functools.partial(
        pl.kernel,
        mesh=mesh,
        out_type=[jax.ShapeDtypeStruct((_NP, _H), _f32)] * 2,
        scratch_types=[
            pltpu.VMEM((_CH,), jnp.int32),
            pltpu.VMEM((_CH,), jnp.int32),
            pltpu.VMEM((_CH, _H), _f32),
            pltpu.VMEM((_RC, _H), _f32),
            pltpu.VMEM_SHARED((_NP, _H), _f32),
            pltpu.SemaphoreType.DMA,
        ],
    )
    def k(hw_hbm, src_hbm, dst_hbm, z_hbm, out0, out1,
          src_v, dst_v, rows_v, stage_v, acc, sem):
        c = lax.axis_index("c")
        s = lax.axis_index("s")
        row0 = s * _RPT
        pltpu.sync_copy(z_hbm, stage_v)
        for i in range(_NRC):
            pltpu.sync_copy(stage_v, acc.at[pl.ds(row0 + i * _RC, _RC)])
        plsc.subcore_barrier()

        base = (s * _NC + c) * _EPW

        def body(j, carry):
            off = pl.multiple_of(base + j * _CH, 8)
            pltpu.sync_copy(src_hbm.at[pl.ds(off, _CH)], src_v)
            pltpu.sync_copy(dst_hbm.at[pl.ds(off, _CH)], dst_v)
            pltpu.async_copy(hw_hbm.at[src_v], rows_v, sem).wait()
            pltpu.sync_copy(rows_v, acc.at[dst_v], add=True)
            return carry

        lax.fori_loop(0, _NCH, body, 0)
        plsc.subcore_barrier()

        @pl.when(c == 0)
        def _():
            for i in range(_NRC):
                r = row0 + i * _RC
                pltpu.sync_copy(acc.at[pl.ds(r, _RC)], stage_v)
                pltpu.sync_copy(stage_v, out0.at[pl.ds(r, _RC)])

        @pl.when(c == 1)
        def _():
            for i in range(_NRC):
                r = row0 + i * _RC
                pltpu.sync_copy(acc.at[pl.ds(r, _RC)], stage_v)
                pltpu.sync_copy(stage_v, out1.at[pl.ds(r, _RC)])

    return k(hw, src, dst, zrows)


def _upd_mid_body(h_ref, a0_ref, a1_ref, wt_ref, wb_ref, b_ref, mw_ref, mb_ref,
                  hn_ref, hw_ref):
    a = a0_ref[...] + a1_ref[...]
    hn = jnp.maximum(
        jnp.dot(h_ref[...], wt_ref[...], preferred_element_type=_f32)
        + jnp.dot(a, wb_ref[...], preferred_element_type=_f32)
        + b_ref[...], 0.0)
    hn_ref[...] = hn
    hw_ref[...] = jnp.dot(hn, mw_ref[...], preferred_element_type=_f32) + mb_ref[...]


def _upd_mid(h, a0, a1, wt, wb, b, mw, mb):
    return pl.pallas_call(
        _upd_mid_body,
        grid=(_N // _R,),
        in_specs=[
            pl.BlockSpec((_R, _H), lambda i: (i, 0)),
            pl.BlockSpec((_R, _H), lambda i: (i, 0)),
            pl.BlockSpec((_R, _H), lambda i: (i, 0)),
            pl.BlockSpec((_H, _H), lambda i: (0, 0)),
            pl.BlockSpec((_H, _H), lambda i: (0, 0)),
            pl.BlockSpec((1, _H), lambda i: (0, 0)),
            pl.BlockSpec((_H, _H), lambda i: (0, 0)),
            pl.BlockSpec((1, _H), lambda i: (0, 0)),
        ],
        out_specs=[
            pl.BlockSpec((_R, _H), lambda i: (i, 0)),
            pl.BlockSpec((_R, _H), lambda i: (i, 0)),
        ],
        out_shape=[jax.ShapeDtypeStruct((_N, _H), _f32)] * 2,
    )(h, a0, a1, wt, wb, b, mw, mb)


def _upd_last_body(h_ref, a0_ref, a1_ref, bt_ref, wt_ref, wb_ref, b_ref, ge_ref):
    a = a0_ref[...] + a1_ref[...]
    hn = jnp.maximum(
        jnp.dot(h_ref[...], wt_ref[...], preferred_element_type=_f32)
        + jnp.dot(a, wb_ref[...], preferred_element_type=_f32)
        + b_ref[...], 0.0)
    gi = lax.broadcasted_iota(jnp.int32, (_R, _G), 1)
    mask = (bt_ref[...] == gi).astype(_f32)
    part = lax.dot_general(mask, hn, (((0,), (0,)), ((), ())),
                           preferred_element_type=_f32)

    @pl.when(pl.program_id(0) == 0)
    def _():
        ge_ref[...] = jnp.zeros_like(ge_ref)

    ge_ref[...] += part


def _upd_last(h, a0, a1, bt, wt, wb, b):
    return pl.pallas_call(
        _upd_last_body,
        grid=(_N // _R,),
        in_specs=[
            pl.BlockSpec((_R, _H), lambda i: (i, 0)),
            pl.BlockSpec((_R, _H), lambda i: (i, 0)),
            pl.BlockSpec((_R, _H), lambda i: (i, 0)),
            pl.BlockSpec((_R, 1), lambda i: (i, 0)),
            pl.BlockSpec((_H, _H), lambda i: (0, 0)),
            pl.BlockSpec((_H, _H), lambda i: (0, 0)),
            pl.BlockSpec((1, _H), lambda i: (0, 0)),
        ],
        out_specs=pl.BlockSpec((_G, _H), lambda i: (0, 0)),
        out_shape=jax.ShapeDtypeStruct((_G, _H), _f32),
    )(h, a0, a1, bt, wt, wb, b)


def _dec_body(ge_ref, mw_ref, mb_ref, lw_ref, lb_ref, eps_ref,
              d1w_ref, d1b_ref, d2w_ref, d2b_ref, nw_ref, nb_ref,
              zm_ref, zl_ref, emb_ref):
    ge = ge_ref[...]
    zm = jnp.dot(ge, mw_ref[...], preferred_element_type=_f32) + mb_ref[...]
    zl = jnp.dot(ge, lw_ref[...], preferred_element_type=_f32) + lb_ref[...]
    zm_ref[...] = zm
    zl_ref[...] = zl
    z = zm + eps_ref[...] * jnp.exp(0.5 * zl)
    g1 = jnp.maximum(
        jnp.dot(z, d1w_ref[...], preferred_element_type=_f32) + d1b_ref[...], 0.0)
    g2 = jnp.maximum(
        jnp.dot(g1, d2w_ref[...], preferred_element_type=_f32) + d2b_ref[...], 0.0)
    emb_ref[...] = jnp.maximum(
        jnp.dot(g2, nw_ref[...], preferred_element_type=_f32) + nb_ref[...], 0.0)


def _full_spec(shape):
    return pl.BlockSpec(shape, lambda: tuple(0 for _ in shape))


def _dec(ge, mw, mb, lw, lb, eps, d1w, d1b, d2w, d2b, nw, nb):
    return pl.pallas_call(
        _dec_body,
        in_specs=[
            _full_spec((_G, _H)), _full_spec((_H, _Z)), _full_spec((1, _Z)),
            _full_spec((_H, _Z)), _full_spec((1, _Z)), _full_spec((_G, _Z)),
            _full_spec((_Z, _H)), _full_spec((1, _H)), _full_spec((_H, _H)),
            _full_spec((1, _H)), _full_spec((_H, _H * _MAXN)),
            _full_spec((1, _H * _MAXN)),
        ],
        out_specs=[_full_spec((_G, _Z)), _full_spec((_G, _Z)),
                   _full_spec((_G, _H * _MAXN))],
        out_shape=[
            jax.ShapeDtypeStruct((_G, _Z), _f32),
            jax.ShapeDtypeStruct((_G, _Z), _f32),
            jax.ShapeDtypeStruct((_G, _H * _MAXN), _f32),
        ],
    )(ge, mw, mb, lw, lb, eps, d1w, d1b, d2w, d2b, nw, nb)


def _nmm_body(e_ref, w_ref, b_ref, o_ref):
    o_ref[...] = (jnp.dot(e_ref[...], w_ref[...], preferred_element_type=_f32)
                  + b_ref[...])


def _nmm(emb2, wcat, bcat):
    rows = _G * _MAXN
    cols = 3 * _H
    return pl.pallas_call(
        _nmm_body,
        in_specs=[_full_spec((rows, _H)), _full_spec((_H, cols)),
                  _full_spec((1, cols))],
        out_specs=_full_spec((rows, cols)),
        out_shape=jax.ShapeDtypeStruct((rows, cols), _f32),
    )(emb2, wcat, bcat)


def _pair_body(u_ref, v_ref, w2t_ref, b2_ref, o_ref):
    u = u_ref[...]
    v = v_ref[...]
    w2t = w2t_ref[...]
    b2 = b2_ref[...]
    lanes = lax.broadcasted_iota(jnp.int32, (1, _MAXN), 1)
    for a in range(_MAXN):
        h1 = jnp.maximum(u[a:a + 1, :] + v, 0.0)
        h2 = jnp.maximum(v[a:a + 1, :] + u, 0.0)
        r1 = lax.dot_general(w2t, h1, (((1,), (1,)), ((), ())),
                             preferred_element_type=_f32)
        r2 = lax.dot_general(w2t, h2, (((1,), (1,)), ((), ())),
                             preferred_element_type=_f32)
        row = (r1 + r2) * 0.5 + b2
        o_ref[a:a + 1, :] = jnp.where(lanes == a, -1e9, row)


def _pair(u, v, w2t, b2):
    return pl.pallas_call(
        _pair_body,
        grid=(_G,),
        in_specs=[
            pl.BlockSpec((None, _MAXN, _H), lambda g: (g, 0, 0)),
            pl.BlockSpec((None, _MAXN, _H), lambda g: (g, 0, 0)),
            pl.BlockSpec((1, _H), lambda g: (0, 0)),
            pl.BlockSpec((1, 1), lambda g: (0, 0)),
        ],
        out_specs=pl.BlockSpec((None, _MAXN, _MAXN), lambda g: (g, 0, 0)),
        out_shape=jax.ShapeDtypeStruct((_G, _MAXN, _MAXN), _f32),
    )(u, v, w2t, b2)


def kernel(x, edge_index, batch, enc_in_W, enc_in_b, msg_W, msg_b, upd_W, upd_b,
           mean_W, mean_b, logvar_W, logvar_b, dec1_W, dec1_b, dec2_W, dec2_b,
           nemb_W, nemb_b, nfeat_W, nfeat_b, edge1_W, edge1_b, edge2_W, edge2_b):
    row = lambda v: v.reshape(1, -1)
    src = edge_index[0]
    dst = edge_index[1]
    zrows = jnp.zeros((_RC, _H), _f32)

    h, hw = _enc(x, enc_in_W, row(enc_in_b), msg_W[0], row(msg_b[0]))

    ge = None
    for l in range(_L):
        a0, a1 = _sc_scatter(hw, src, dst, zrows)
        wt = upd_W[l][:_H]
        wb = upd_W[l][_H:]
        if l + 1 < _L:
            h, hw = _upd_mid(h, a0, a1, wt, wb, row(upd_b[l]),
                             msg_W[l + 1], row(msg_b[l + 1]))
        else:
            ge = _upd_last(h, a0, a1, batch.reshape(_N, 1), wt, wb,
                           row(upd_b[l]))

    eps = jax.random.normal(jax.random.key(42), (_G, _Z), _f32)
    z_mean, z_logvar, emb_flat = _dec(
        ge, mean_W, row(mean_b), logvar_W, row(logvar_b), eps,
        dec1_W, row(dec1_b), dec2_W, row(dec2_b), nemb_W, row(nemb_b))

    emb2 = emb_flat.reshape(_G * _MAXN, _H)
    wcat = jnp.concatenate([nfeat_W, edge1_W[:_H], edge1_W[_H:]], axis=1)
    bcat = jnp.concatenate(
        [nfeat_b, edge1_b, jnp.zeros((_H,), _f32)]).reshape(1, 3 * _H)
    out = _nmm(emb2, wcat, bcat)

    node_features = out[:, :_H].reshape(_G, _MAXN, _D)
    u = out[:, _H:2 * _H].reshape(_G, _MAXN, _H)
    v = out[:, 2 * _H:].reshape(_G, _MAXN, _H)

    adj_logits = _pair(u, v, edge2_W.reshape(1, _H), edge2_b.reshape(1, 1))
    return (node_features, adj_logits, z_mean, z_logvar)
